# baseline (speedup 1.0000x reference)
import jax
import jax.numpy as jnp
from jax.experimental import pallas as pl
from jax.experimental.pallas import tpu as pltpu

_BM = 200


def _support_kernel(v_ref, w_ref, support_ref):
    support = jnp.dot(v_ref[...], w_ref[...],
                      preferred_element_type=jnp.float32)
    support_ref[...] = support.astype(jnp.bfloat16)


def _spmm_kernel(adj_ref, support_ref, out_ref):
    adj_bf = adj_ref[...].astype(jnp.bfloat16)
    out_ref[...] = jnp.dot(adj_bf, support_ref[...],
                           preferred_element_type=jnp.float32)


def kernel(v, adj, W):
    n, d_in = v.shape
    d_out = W.shape[1]
    support = pl.pallas_call(
        _support_kernel,
        out_shape=jax.ShapeDtypeStruct((n, d_out), jnp.bfloat16),
    )(v, W)
    bm = _BM if n % _BM == 0 else n
    out = pl.pallas_call(
        _spmm_kernel,
        grid=(n // bm,),
        in_specs=[
            pl.BlockSpec((bm, n), lambda i: (i, 0)),
            pl.BlockSpec((n, d_out), lambda i: (0, 0)),
        ],
        out_specs=pl.BlockSpec((bm, d_out), lambda i: (i, 0)),
        out_shape=jax.ShapeDtypeStruct((n, d_out), jnp.float32),
        compiler_params=pltpu.CompilerParams(
            dimension_semantics=("parallel",)),
    )(adj, support)
    return (out, adj)


# speedup vs baseline: 1.0107x; 1.0107x over previous
import jax
import jax.numpy as jnp
from jax.experimental import pallas as pl
from jax.experimental.pallas import tpu as pltpu

_BM = 400


def _gcn_kernel(v_ref, w_ref, adj_ref, out_ref, support_ref):
    @pl.when(pl.program_id(0) == 0)
    def _():
        support = jnp.dot(v_ref[...], w_ref[...],
                          preferred_element_type=jnp.float32)
        support_ref[...] = support.astype(jnp.bfloat16)

    adj_bf = adj_ref[...].astype(jnp.bfloat16)
    out_ref[...] = jnp.dot(adj_bf, support_ref[...],
                           preferred_element_type=jnp.float32)


def kernel(v, adj, W):
    n, d_in = v.shape
    d_out = W.shape[1]
    bm = _BM if n % _BM == 0 else n
    out = pl.pallas_call(
        _gcn_kernel,
        grid=(n // bm,),
        in_specs=[
            pl.BlockSpec((n, d_in), lambda i: (0, 0)),
            pl.BlockSpec((d_in, d_out), lambda i: (0, 0)),
            pl.BlockSpec((bm, n), lambda i: (i, 0)),
        ],
        out_specs=pl.BlockSpec((bm, d_out), lambda i: (i, 0)),
        out_shape=jax.ShapeDtypeStruct((n, d_out), jnp.float32),
        scratch_shapes=[pltpu.VMEM((n, d_out), jnp.bfloat16)],
    )(v, W, adj)
    return (out, adj)
